# baseline (speedup 1.0000x reference)
import functools

import jax
import jax.numpy as jnp
from jax.experimental import pallas as pl

B = 256
NN = 5
H = 50
CATN = 18
C = CATN + 1
D = 400
AD = 128
NODES = H + CATN
L = 2
R = NN * C
BB = 4

_INV_SCALE = 1.0 / (AD ** 0.5)


def _dot(x, w):
    return jax.lax.dot_general(x, w, (((x.ndim - 1,), (0,)), ((), ())),
                               preferred_element_type=jnp.float32)


def _sue_kernel(h0_ref, cand_ref, graph_ref, maskf_ref, idx_ref,
                Wg_ref, bg_ref, WK_ref, WQ_ref, bQ_ref,
                Waff_ref, baff_ref, Wck_ref, Wcq_ref, bcq_ref, out_ref):
    h0 = h0_ref[...]
    g = h0
    for l in range(L):
        agg = jnp.stack([jnp.dot(graph_ref[u], g[u],
                                 preferred_element_type=jnp.float32)
                         for u in range(BB)])
        g = g + jax.nn.relu(_dot(agg, Wg_ref[l]) + bg_ref[l])
    gf = (g + h0)[:, :H, :]

    K3 = _dot(gf, WK_ref[...])
    cand = cand_ref[...]
    Q3 = _dot(cand, WQ_ref[...]) + bQ_ref[...]
    Qc3 = _dot(cand, Wcq_ref[...]) + bcq_ref[...]

    row_iota = jax.lax.broadcasted_iota(jnp.int32, (R, NN), 0)
    col_iota = jax.lax.broadcasted_iota(jnp.int32, (R, NN), 1)
    Pn = (row_iota // C == col_iota).astype(jnp.float32)
    rowc_iota = jax.lax.broadcasted_iota(jnp.int32, (R, C), 0)
    cc_iota = jax.lax.broadcasted_iota(jnp.int32, (R, C), 1)
    Pc = (rowc_iota % C == cc_iota).astype(jnp.float32)
    nmask = (jax.lax.broadcasted_iota(jnp.int32, (NN, R), 0)
             == jax.lax.broadcasted_iota(jnp.int32, (NN, R), 1) // C
             ).astype(jnp.float32)

    intra_list = []
    for u in range(BB):
        a = jax.lax.dot_general(
            Q3[u], K3[u], (((1,), (1,)), ((), ())),
            preferred_element_type=jnp.float32) * _INV_SCALE

        idx_u = idx_ref[u]
        cat_iota = jax.lax.broadcasted_iota(jnp.int32, (C, H), 0)
        onehot = (cat_iota == idx_u).astype(jnp.float32)

        masked = jnp.where(onehot[None, :, :] > 0, a[:, None, :], -1e30)
        M = jnp.max(masked, axis=-1)
        m_h = jnp.dot(M, onehot, preferred_element_type=jnp.float32)
        ex = jnp.exp(a - m_h)
        ssum = jax.lax.dot_general(
            ex, onehot, (((1,), (1,)), ((), ())),
            preferred_element_type=jnp.float32)
        denom = jnp.dot(ssum, onehot,
                        preferred_element_type=jnp.float32) + 1e-12
        alpha = ex / denom

        cfull = ((jax.lax.broadcasted_iota(jnp.int32, (R, H), 0) % C)
                 == idx_u).astype(jnp.float32)
        alphaR = jnp.dot(Pn, alpha,
                         preferred_element_type=jnp.float32)
        intra_list.append(jnp.dot(cfull * alphaR, gf[u],
                                  preferred_element_type=jnp.float32))

    intra = jnp.stack(intra_list)
    intra2 = jax.nn.relu(_dot(intra, Waff_ref[...]) + baff_ref[...]) + intra
    Kc3 = _dot(intra2, Wck_ref[...])

    for u in range(BB):
        E = jax.lax.dot_general(
            Qc3[u], Kc3[u], (((1,), (1,)), ((), ())),
            preferred_element_type=jnp.float32)
        e = jnp.dot(E * nmask, Pc,
                    preferred_element_type=jnp.float32) * _INV_SCALE
        e = jnp.where(maskf_ref[u] > 0, e, -1e9)
        e = e - jnp.max(e, axis=-1, keepdims=True)
        we = jnp.exp(e)
        w = we / jnp.sum(we, axis=-1, keepdims=True)
        wR = jax.lax.dot_general(
            w, Pc, (((1,), (1,)), ((), ())),
            preferred_element_type=jnp.float32) * nmask
        out_ref[u] = jnp.dot(wR, intra2[u],
                             preferred_element_type=jnp.float32)


@jax.jit
def _sue_pallas(h0, cand, graph, maskf, idx, W_gcn, b_gcn, W_K,
                W_Q, b_Q, W_aff, b_aff, W_ck, W_cq, b_cq):
    grid = (B // BB,)
    data_spec3 = lambda s1, s2: pl.BlockSpec((BB, s1, s2), lambda i: (i, 0, 0))
    w_spec = lambda shape: pl.BlockSpec(shape, lambda i: (0,) * len(shape))
    return pl.pallas_call(
        _sue_kernel,
        grid=grid,
        in_specs=[
            data_spec3(NODES, D),
            data_spec3(NN, D),
            data_spec3(NODES, NODES),
            data_spec3(1, C),
            data_spec3(1, H),
            w_spec((L, D, D)),
            w_spec((L, 1, D)),
            w_spec((D, AD)),
            w_spec((D, AD)),
            w_spec((1, AD)),
            w_spec((D, D)),
            w_spec((1, D)),
            w_spec((D, AD)),
            w_spec((D, AD)),
            w_spec((1, AD)),
        ],
        out_specs=data_spec3(NN, D),
        out_shape=jax.ShapeDtypeStruct((B, NN, D), jnp.float32),
    )(h0, cand, graph, maskf, idx, W_gcn, b_gcn, W_K, W_Q, b_Q,
      W_aff, b_aff, W_ck, W_cq, b_cq)


def kernel(history_embedding, candidate_news_representation, user_history_graph,
           user_history_category_mask, user_history_category_indices,
           proxy_node_embedding, W_gcn, b_gcn, W_K, W_Q, b_Q, W_aff, b_aff,
           W_ck, W_cq, b_cq):
    h0 = jnp.concatenate(
        [history_embedding,
         jnp.broadcast_to(proxy_node_embedding[None], (B, CATN, D))], axis=1)
    maskf = user_history_category_mask.at[:, -1].set(1)
    maskf = (maskf > 0).astype(jnp.float32).reshape(B, 1, C)
    idx = user_history_category_indices.astype(jnp.int32).reshape(B, 1, H)
    return _sue_pallas(
        h0, candidate_news_representation, user_history_graph,
        maskf, idx, W_gcn, b_gcn.reshape(L, 1, D), W_K, W_Q,
        b_Q.reshape(1, AD), W_aff, b_aff.reshape(1, D), W_ck, W_cq,
        b_cq.reshape(1, AD))
